# Initial kernel scaffold; baseline (speedup 1.0000x reference)
import jax
import jax.numpy as jnp
from jax.experimental import pallas as pl


def kernel(node_features, edge_index, edge_features, batch_indices, W_enc, b_enc, W_e1, b_e1, W_e2, b_e2, W_ih, W_hh, b_ih, b_hh, W_li, W_lh, b_l, W_o1, b_o1, W_o2, b_o2):
    raise NotImplementedError("write your pallas kernel here")


# speedup vs baseline: 2.8657x; 2.8657x over previous
import functools

import jax
import jax.numpy as jnp
import numpy as np
from jax import lax
from jax.experimental import pallas as pl
from jax.experimental.pallas import tpu as pltpu
from jax.experimental.pallas import tpu_sc as plsc

N_NODES = 10000
N_EDGES = 160000
HIDDEN = 32
NUM_GRAPHS = 64
N_MESSAGES = 3
SET2SET_STEPS = 4

NC = 2
NS = 16
NW = NC * NS

SCAT_CH = 128
SCAT_NCH = 40
E_PER_W = SCAT_CH * SCAT_NCH
E_PAD = NW * E_PER_W
GATH_CH = 1024
GATH_NCH = E_PER_W // GATH_CH
DUMP = N_NODES
N_ACC = N_NODES + 240
ACC_PER_TILE = N_ACC // NS

_H = HIDDEN
_P_SEL = np.repeat(np.eye(_H, dtype=np.float32), _H, axis=0)


def _mesh():
    return plsc.VectorSubcoreMesh(
        core_axis_name="c", subcore_axis_name="s", num_cores=NC, num_subcores=NS
    )


def _sc_gather(h, src_pad):
    def body(h_hbm, idx_hbm, out_hbm, idx_v, rows_v, sem):
        c = lax.axis_index("c")
        s = lax.axis_index("s")
        wid = s * NC + c
        base = wid * E_PER_W

        def chunk(i, _):
            off = base + i * GATH_CH
            pltpu.sync_copy(idx_hbm.at[pl.ds(off, GATH_CH)], idx_v)
            pltpu.async_copy(h_hbm.at[idx_v], rows_v, sem).wait()
            pltpu.sync_copy(rows_v, out_hbm.at[pl.ds(off, GATH_CH)])
            return 0

        lax.fori_loop(0, GATH_NCH, chunk, 0)

    k = pl.kernel(
        body,
        out_type=jax.ShapeDtypeStruct((E_PAD, _H), jnp.float32),
        mesh=_mesh(),
        compiler_params=pltpu.CompilerParams(use_tc_tiling_on_sc=False),
        scratch_types=[
            pltpu.VMEM((GATH_CH,), jnp.int32),
            pltpu.VMEM((GATH_CH, _H), jnp.float32),
            pltpu.SemaphoreType.DMA,
        ],
    )
    return k(h, src_pad)


def _sc_scatter(me, dst3):
    zeros = jnp.zeros((ACC_PER_TILE, _H), jnp.float32)

    def body(me_hbm, idx_hbm, z_hbm, out_hbm, idx_v, m_v, acc_sh):
        c = lax.axis_index("c")
        s = lax.axis_index("s")
        wid = s * NC + c
        base = wid * E_PER_W
        row0 = s * ACC_PER_TILE
        pltpu.sync_copy(z_hbm, acc_sh.at[pl.ds(row0, ACC_PER_TILE)])
        pltpu.sync_copy(idx_hbm.at[wid], idx_v)
        plsc.subcore_barrier()

        def chunk(j, _):
            pltpu.sync_copy(me_hbm.at[pl.ds(base + j * SCAT_CH, SCAT_CH)], m_v)
            pltpu.sync_copy(m_v, acc_sh.at[idx_v.at[j]], add=True)
            return 0

        lax.fori_loop(0, SCAT_NCH, chunk, 0)
        plsc.subcore_barrier()
        pltpu.sync_copy(
            acc_sh.at[pl.ds(row0, ACC_PER_TILE)],
            out_hbm.at[c, pl.ds(row0, ACC_PER_TILE)],
        )

    k = pl.kernel(
        body,
        out_type=jax.ShapeDtypeStruct((NC, N_ACC, _H), jnp.float32),
        mesh=_mesh(),
        compiler_params=pltpu.CompilerParams(use_tc_tiling_on_sc=False),
        scratch_types=[
            pltpu.VMEM((SCAT_NCH, SCAT_CH), jnp.int32),
            pltpu.VMEM((SCAT_CH, _H), jnp.float32),
            pltpu.VMEM_SHARED((N_ACC, _H), jnp.float32),
        ],
    )
    return k(me, dst3, zeros)


def _encode(x, w, b):
    def body(x_ref, w_ref, b_ref, o_ref):
        o_ref[:] = jnp.maximum(
            jnp.dot(x_ref[:], w_ref[:], preferred_element_type=jnp.float32)
            + b_ref[:],
            0.0,
        )

    return pl.pallas_call(
        body, out_shape=jax.ShapeDtypeStruct((N_NODES, _H), jnp.float32)
    )(x, w, b)


def _edge_mlp(ef_pad, w1, b1):
    BE1 = 8192
    grid = (E_PAD // BE1,)

    def body(x_ref, w_ref, b_ref, o_ref):
        o_ref[:] = jnp.maximum(
            jnp.dot(x_ref[:], w_ref[:], preferred_element_type=jnp.float32)
            + b_ref[:],
            0.0,
        )

    return pl.pallas_call(
        body,
        grid=grid,
        in_specs=[
            pl.BlockSpec((BE1, 16), lambda i: (i, 0)),
            pl.BlockSpec((16, 128), lambda i: (0, 0)),
            pl.BlockSpec((1, 128), lambda i: (0, 0)),
        ],
        out_specs=pl.BlockSpec((BE1, 128), lambda i: (i, 0)),
        out_shape=jax.ShapeDtypeStruct((E_PAD, 128), jnp.float32),
    )(ef_pad, w1, b1)


def _message(u, hs, w2, be2, psel):
    BE = 512
    grid = (E_PAD // BE,)

    def body(u_ref, hs_ref, w2_ref, b_ref, p_ref, o_ref):
        a = (
            jnp.dot(u_ref[:], w2_ref[:], preferred_element_type=jnp.float32)
            + b_ref[:]
        )
        ht = jnp.concatenate([hs_ref[:]] * _H, axis=1)
        o_ref[:] = jnp.dot(a * ht, p_ref[:], preferred_element_type=jnp.float32)

    return pl.pallas_call(
        body,
        grid=grid,
        in_specs=[
            pl.BlockSpec((BE, 128), lambda i: (i, 0)),
            pl.BlockSpec((BE, _H), lambda i: (i, 0)),
            pl.BlockSpec((128, _H * _H), lambda i: (0, 0)),
            pl.BlockSpec((1, _H * _H), lambda i: (0, 0)),
            pl.BlockSpec((_H * _H, _H), lambda i: (0, 0)),
        ],
        out_specs=pl.BlockSpec((BE, _H), lambda i: (i, 0)),
        out_shape=jax.ShapeDtypeStruct((E_PAD, _H), jnp.float32),
    )(u, hs, w2, be2, psel)


def _gru(m2, h, w_ih, w_hh, b_ih, b_hh):
    def body(m_ref, h_ref, wi_ref, wh_ref, bi_ref, bh_ref, o_ref):
        m = m_ref[0, :N_NODES, :] + m_ref[1, :N_NODES, :]
        h = h_ref[:]
        gi = jnp.dot(m, wi_ref[:], preferred_element_type=jnp.float32) + bi_ref[:]
        gh = jnp.dot(h, wh_ref[:], preferred_element_type=jnp.float32) + bh_ref[:]
        i_r, i_z, i_n = gi[:, :_H], gi[:, _H : 2 * _H], gi[:, 2 * _H :]
        h_r, h_z, h_n = gh[:, :_H], gh[:, _H : 2 * _H], gh[:, 2 * _H :]
        r = jax.nn.sigmoid(i_r + h_r)
        z = jax.nn.sigmoid(i_z + h_z)
        n = jnp.tanh(i_n + r * h_n)
        o_ref[:] = (1.0 - z) * n + z * h

    return pl.pallas_call(
        body, out_shape=jax.ShapeDtypeStruct((N_NODES, _H), jnp.float32)
    )(m2, h, w_ih, w_hh, b_ih, b_hh)


def _set2set(h, batch2d, w_li, w_lh, b_l, w_o1, b_o1, w_o2, b_o2):
    G = NUM_GRAPHS

    def body(h_ref, bi_ref, wli_ref, wlh_ref, bl_ref, wo1_ref, bo1_ref,
             wo2_ref, bo2_ref, o_ref):
        h = h_ref[:]
        gid = lax.broadcasted_iota(jnp.int32, (G, N_NODES), 0)
        mask = bi_ref[:] == gid
        maskf = mask.astype(jnp.float32)
        q_star = jnp.zeros((G, 2 * _H), jnp.float32)
        lh = jnp.zeros((G, _H), jnp.float32)
        lc = jnp.zeros((G, _H), jnp.float32)
        for _ in range(SET2SET_STEPS):
            gates = (
                jnp.dot(q_star, wli_ref[:], preferred_element_type=jnp.float32)
                + jnp.dot(lh, wlh_ref[:], preferred_element_type=jnp.float32)
                + bl_ref[:]
            )
            gi_ = gates[:, :_H]
            gf_ = gates[:, _H : 2 * _H]
            gg_ = gates[:, 2 * _H : 3 * _H]
            go_ = gates[:, 3 * _H :]
            lc = jax.nn.sigmoid(gf_) * lc + jax.nn.sigmoid(gi_) * jnp.tanh(gg_)
            lh = jax.nn.sigmoid(go_) * jnp.tanh(lc)
            q = lh
            st = lax.dot_general(
                q, h, (((1,), (1,)), ((), ())),
                preferred_element_type=jnp.float32,
            )
            stm = jnp.where(mask, st, -1e30)
            emax = jnp.max(stm, axis=1, keepdims=True)
            emax = jnp.where(emax > -1e29, emax, 0.0)
            ex = jnp.exp(stm - emax)
            denom = jnp.sum(ex, axis=1, keepdims=True)
            aw = ex / jnp.where(denom > 0.0, denom, 1.0)
            r_read = jnp.dot(aw, h, preferred_element_type=jnp.float32)
            q_star = jnp.concatenate([q, r_read], axis=1)
        hid = jnp.maximum(
            jnp.dot(q_star, wo1_ref[:], preferred_element_type=jnp.float32)
            + bo1_ref[:],
            0.0,
        )
        o_ref[:] = (
            jnp.dot(hid, wo2_ref[:], preferred_element_type=jnp.float32)
            + bo2_ref[:]
        )

    return pl.pallas_call(
        body, out_shape=jax.ShapeDtypeStruct((G, 3), jnp.float32)
    )(h, batch2d, w_li, w_lh, b_l, w_o1, b_o1, w_o2, b_o2)


def kernel(node_features, edge_index, edge_features, batch_indices, W_enc,
           b_enc, W_e1, b_e1, W_e2, b_e2, W_ih, W_hh, b_ih, b_hh, W_li, W_lh,
           b_l, W_o1, b_o1, W_o2, b_o2):
    src = edge_index[0]
    dst = edge_index[1]
    pad = E_PAD - N_EDGES
    src_pad = jnp.concatenate([src, jnp.zeros((pad,), jnp.int32)])
    dst3 = jnp.concatenate([dst, jnp.full((pad,), DUMP, jnp.int32)]).reshape(
        NW, SCAT_NCH, SCAT_CH
    )
    ef_pad = jnp.concatenate(
        [edge_features, jnp.zeros((pad, edge_features.shape[1]), jnp.float32)]
    )
    psel = jnp.asarray(_P_SEL)
    b_enc2 = b_enc.reshape(1, -1)
    b_e12 = b_e1.reshape(1, -1)
    b_e22 = b_e2.reshape(1, -1)
    b_ih2 = b_ih.reshape(1, -1)
    b_hh2 = b_hh.reshape(1, -1)
    b_l2 = b_l.reshape(1, -1)
    b_o12 = b_o1.reshape(1, -1)
    b_o22 = b_o2.reshape(1, -1)
    batch2d = batch_indices.reshape(1, N_NODES)

    h = _encode(node_features, W_enc, b_enc2)
    u = _edge_mlp(ef_pad, W_e1, b_e12)
    for _ in range(N_MESSAGES):
        hs = _sc_gather(h, src_pad)
        me = _message(u, hs, W_e2, b_e22, psel)
        m2 = _sc_scatter(me, dst3)
        h = _gru(m2, h, W_ih, W_hh, b_ih2, b_hh2)
    return _set2set(h, batch2d, W_li, W_lh, b_l2, W_o1, b_o1.reshape(1, -1),
                    W_o2, b_o22)
